# baseline (speedup 1.0000x reference)
import functools

import jax
import jax.numpy as jnp
import numpy as np
from jax import lax
from jax.experimental import pallas as pl
from jax.experimental.pallas import tpu as pltpu
from jax.experimental.pallas import tpu_sc as plsc

_CONTEXT = 2
_EMB = 64
_HEADS = 4
_HDIM = _EMB // _HEADS
_EPS = 1e-5

_NC = 2
_NS = 16
_NW = _NC * _NS
_CH = 96
_LANES = 16

_BLOCKDIAG = np.kron(np.eye(_HEADS, dtype=np.float32),
                     np.ones((_HDIM, _HDIM), dtype=np.float32))


_PH = 501760


def _tc_pack(table_t):
    d, v = table_t.shape
    blk = 2048
    nblk = _PH // blk

    def body(a_ref, b_ref, out_ref):
        eye = jnp.eye(_EMB, dtype=jnp.float32)
        dn = (((0,), (0,)), ((), ()))
        out_ref[:, 0:_EMB] = lax.dot_general(
            a_ref[...], eye, dn, precision=lax.Precision.HIGHEST,
            preferred_element_type=jnp.float32)
        out_ref[:, _EMB:128] = lax.dot_general(
            b_ref[...], eye, dn, precision=lax.Precision.HIGHEST,
            preferred_element_type=jnp.float32)

    return pl.pallas_call(
        body,
        grid=(nblk,),
        in_specs=[
            pl.BlockSpec((d, blk), lambda i: (0, i)),
            pl.BlockSpec((d, blk),
                         lambda i: (0, jnp.minimum(i + nblk, (v - 1) // blk))),
        ],
        out_specs=pl.BlockSpec((blk, 128), lambda i: (i, 0)),
        out_shape=jax.ShapeDtypeStruct((_PH, 128), jnp.float32),
        compiler_params=pltpu.CompilerParams(
            fuse_transposed_lhs_in_matmul=True),
    )(table_t, table_t)


def _sc_gather(tab2, idx3f, idx3h, n):
    _, nch, ch = idx3f.shape
    per_w = nch * ch
    mesh = plsc.VectorSubcoreMesh(core_axis_name="c", subcore_axis_name="s")

    @functools.partial(
        pl.kernel,
        mesh=mesh,
        out_type=jax.ShapeDtypeStruct((n, _EMB), jnp.float32),
        scratch_types=[
            pltpu.VMEM((nch, ch), jnp.int32),
            pltpu.VMEM((nch, ch), jnp.int32),
            pltpu.VMEM((ch, 128), jnp.float32),
            pltpu.VMEM((ch, 128), jnp.float32),
            pltpu.VMEM((ch, _EMB), jnp.float32),
            pltpu.VMEM((ch, _EMB), jnp.float32),
            pltpu.SemaphoreType.DMA,
            pltpu.SemaphoreType.DMA,
            pltpu.SemaphoreType.DMA,
        ],
        compiler_params=pltpu.CompilerParams(use_tc_tiling_on_sc=False),
    )
    def k(tab_hbm, idxf_hbm, idxh_hbm, out_hbm, idxf_v, idxh_v,
          stage0, stage1, rows0, rows1, sem0, sem1, osem):
        wid = lax.axis_index("s") * _NC + lax.axis_index("c")
        base = wid * per_w
        pltpu.sync_copy(idxf_hbm.at[wid], idxf_v)
        pltpu.sync_copy(idxh_hbm.at[wid], idxh_v)

        def extract(i, stage, rows):
            def group(g, c2):
                v16 = idxf_v[i, pl.ds(g * _LANES, _LANES)]
                for lane in range(_LANES):
                    j = g * _LANES + lane
                    h = (v16[lane] >= _PH).astype(jnp.int32) * _EMB
                    for kk in range(_EMB // _LANES):
                        rows[j, pl.ds(kk * _LANES, _LANES)] = (
                            stage[j, pl.ds(h + kk * _LANES, _LANES)])
                return c2
            lax.fori_loop(0, ch // _LANES, group, 0)

        def body(i, carry):
            pltpu.async_copy(tab_hbm.at[idxh_v.at[i]], stage0, sem0).wait()
            extract(i, stage0, rows0)
            pltpu.async_copy(
                rows0, out_hbm.at[pl.ds(base + i * ch, ch)], osem).wait()
            return carry

        lax.fori_loop(0, nch, body, 0)

    return k(tab2, idx3f, idx3h)


def _tc_compute(embs3, amatt):
    u1, b, _ = embs3.shape
    u = u1 - 1

    def body(emb_ref, a_ref, out_ref, prev_ref):
        j = pl.program_id(0)
        zt = jnp.transpose(emb_ref[0])

        @pl.when(j > 0)
        def _():
            z0 = prev_ref[...]
            z1 = zt
            w0 = jnp.dot(a_ref[0], z0, preferred_element_type=jnp.float32)
            w1 = jnp.dot(a_ref[1], z1, preferred_element_type=jnp.float32)
            t = z0 * w0 + z1 * w1
            mean = jnp.mean(t, axis=0, keepdims=True)
            c = t - mean
            var = jnp.mean(c * c, axis=0, keepdims=True)
            out_ref[0] = c * lax.rsqrt(var + _EPS)

        prev_ref[...] = zt

    return pl.pallas_call(
        body,
        grid=(u1,),
        in_specs=[
            pl.BlockSpec((1, b, _EMB), lambda j: (j, 0, 0)),
            pl.BlockSpec((2, _EMB, _EMB), lambda j: (0, 0, 0)),
        ],
        out_specs=pl.BlockSpec((1, _EMB, b), lambda j: (jnp.maximum(j - 1, 0), 0, 0)),
        out_shape=jax.ShapeDtypeStruct((u, _EMB, b), jnp.float32),
        scratch_shapes=[pltpu.VMEM((_EMB, b), jnp.float32)],
    )(embs3, amatt)


def kernel(y, table, pos_embs):
    b, u = y.shape
    blanks = jnp.zeros((_CONTEXT - 1, b), dtype=y.dtype)
    appended_t = jnp.concatenate([blanks, y.T], axis=0)
    n = b * (u + 1)
    per_w = n // _NW
    nch = per_w // _CH
    idx3f = appended_t.reshape(_NW, nch, _CH)
    idx3h = jnp.where(idx3f < _PH, idx3f, idx3f - _PH)

    tab2 = _tc_pack(table.T)
    embs = _sc_gather(tab2, idx3f, idx3h, n)
    embs3 = embs.reshape(u + 1, b, _EMB)

    posv = jnp.transpose(pos_embs, (2, 0, 1)).reshape(_CONTEXT, _EMB)
    amatt = jnp.asarray(_BLOCKDIAG)[None] * posv[:, None, :]

    p = _tc_compute(embs3, amatt)
    out = jnp.transpose(p, (2, 0, 1))
    state = y[:, u - (_CONTEXT - 1):]
    return out, state


# speedup vs baseline: 1.1076x; 1.1076x over previous
import functools

import jax
import jax.numpy as jnp
import numpy as np
from jax import lax
from jax.experimental import pallas as pl
from jax.experimental.pallas import tpu as pltpu
from jax.experimental.pallas import tpu_sc as plsc

_CONTEXT = 2
_EMB = 64
_HEADS = 4
_HDIM = _EMB // _HEADS
_EPS = 1e-5

_NC = 2
_NS = 16
_NW = _NC * _NS
_CH = 48
_LANES = 16

_BLOCKDIAG = np.kron(np.eye(_HEADS, dtype=np.float32),
                     np.ones((_HDIM, _HDIM), dtype=np.float32))


_PWPAD = 7168


def _sc_gather(table, idxp, n, per_w):
    nch = per_w // _CH
    mesh = plsc.VectorSubcoreMesh(core_axis_name="c", subcore_axis_name="s")

    @functools.partial(
        pl.kernel,
        mesh=mesh,
        out_type=jax.ShapeDtypeStruct((n, _EMB), jnp.float32),
        scratch_types=[
            pltpu.VMEM((_PWPAD,), jnp.int32),
            pltpu.VMEM((_CH, 8, _EMB), jnp.float32),
            pltpu.VMEM((_CH, 8, _EMB), jnp.float32),
            pltpu.VMEM((_CH, _EMB), jnp.float32),
            pltpu.VMEM((_CH, _EMB), jnp.float32),
            pltpu.SemaphoreType.DMA,
            pltpu.SemaphoreType.DMA,
            pltpu.SemaphoreType.DMA,
        ],
        compiler_params=pltpu.CompilerParams(use_tc_tiling_on_sc=True),
    )
    def k(table_hbm, idx_hbm, out_hbm, idx_v, stage0, stage1, rows0, rows1,
          sem0, sem1, osem):
        wid = lax.axis_index("s") * _NC + lax.axis_index("c")
        base = wid * per_w
        pltpu.sync_copy(
            idx_hbm.at[pl.ds(pl.multiple_of(wid * _PWPAD, 1024), _PWPAD)],
            idx_v)

        def each_dma(i, stage, sem, fn):
            def group(g, c2):
                v16 = idx_v[pl.ds(i * _CH + g * _LANES, _LANES)]
                for lane in range(_LANES):
                    v = v16[lane]
                    v0 = pl.multiple_of((v // 8) * 8, 8)
                    fn(pltpu.make_async_copy(table_hbm.at[pl.ds(v0, 8)],
                                             stage.at[g * _LANES + lane], sem))
                return c2
            lax.fori_loop(0, _CH // _LANES, group, 0)

        def enqueue(i, stage, sem):
            each_dma(i, stage, sem, lambda d: d.start())

        def drain(i, stage, sem):
            each_dma(i, stage, sem, lambda d: d.wait())

        def extract(i, stage, rows):
            def group(g, c2):
                v16 = idx_v[pl.ds(i * _CH + g * _LANES, _LANES)]
                for lane in range(_LANES):
                    j = g * _LANES + lane
                    v = v16[lane]
                    s = v - (v // 8) * 8
                    for kk in range(_EMB // _LANES):
                        rows[j, pl.ds(kk * _LANES, _LANES)] = (
                            stage[j, s, pl.ds(kk * _LANES, _LANES)])
                return c2
            lax.fori_loop(0, _CH // _LANES, group, 0)

        enqueue(0, stage0, sem0)

        def body(i, carry):
            even = lax.rem(i, 2) == 0

            def do(stage, sem, stage_n, sem_n, rows):
                @pl.when(i + 1 < nch)
                def _():
                    enqueue(i + 1, stage_n, sem_n)

                drain(i, stage, sem)
                extract(i, stage, rows)
                pltpu.async_copy(
                    rows, out_hbm.at[pl.ds(base + i * _CH, _CH)], osem).wait()

            @pl.when(even)
            def _():
                do(stage0, sem0, stage1, sem1, rows0)

            @pl.when(jnp.logical_not(even))
            def _():
                do(stage1, sem1, stage0, sem0, rows1)

            return carry

        lax.fori_loop(0, nch, body, 0)

    return k(table, idxp)


def _tc_compute(embs3, amatt):
    u1, b, _ = embs3.shape
    u = u1 - 1

    def body(emb_ref, a_ref, out_ref, prev_ref):
        j = pl.program_id(0)
        zt = jnp.transpose(emb_ref[0])

        @pl.when(j > 0)
        def _():
            z0 = prev_ref[...]
            z1 = zt
            w0 = jnp.dot(a_ref[0], z0, preferred_element_type=jnp.float32)
            w1 = jnp.dot(a_ref[1], z1, preferred_element_type=jnp.float32)
            t = z0 * w0 + z1 * w1
            mean = jnp.mean(t, axis=0, keepdims=True)
            c = t - mean
            var = jnp.mean(c * c, axis=0, keepdims=True)
            out_ref[0] = c * lax.rsqrt(var + _EPS)

        prev_ref[...] = zt

    return pl.pallas_call(
        body,
        grid=(u1,),
        in_specs=[
            pl.BlockSpec((1, b, _EMB), lambda j: (j, 0, 0)),
            pl.BlockSpec((2, _EMB, _EMB), lambda j: (0, 0, 0)),
        ],
        out_specs=pl.BlockSpec((1, _EMB, b), lambda j: (jnp.maximum(j - 1, 0), 0, 0)),
        out_shape=jax.ShapeDtypeStruct((u, _EMB, b), jnp.float32),
        scratch_shapes=[pltpu.VMEM((_EMB, b), jnp.float32)],
    )(embs3, amatt)


def kernel(y, table, pos_embs):
    b, u = y.shape
    blanks = jnp.zeros((_CONTEXT - 1, b), dtype=y.dtype)
    appended_t = jnp.concatenate([blanks, y.T], axis=0)
    n = b * (u + 1)
    per_w = n // _NW
    idx2 = appended_t.reshape(_NW, per_w)
    idxp = jnp.pad(idx2, ((0, 0), (0, _PWPAD - per_w))).reshape(-1)

    embs = _sc_gather(table, idxp, n, per_w)
    embs3 = embs.reshape(u + 1, b, _EMB)

    posv = jnp.transpose(pos_embs, (2, 0, 1)).reshape(_CONTEXT, _EMB)
    amatt = jnp.asarray(_BLOCKDIAG)[None] * posv[:, None, :]

    p = _tc_compute(embs3, amatt)
    out = jnp.transpose(p, (2, 0, 1))
    state = y[:, u - (_CONTEXT - 1):]
    return out, state
